# baseline (speedup 1.0000x reference)
import jax
import jax.numpy as jnp
from jax.experimental import pallas as pl
from jax.experimental.pallas import tpu as pltpu


def _body(l1_ref, l2_ref, fk_ref, w1t_ref, w2t_ref, b_ref, out_ref):
    hm = l1_ref.shape[0]
    fk = fk_ref[...]
    w1t = w1t_ref[...]
    w2t = w2t_ref[...]
    b = b_ref[...]
    m = pl.program_id(0)
    for i, l_ref in enumerate((l1_ref, l2_ref)):
        x = jnp.dot(l_ref[...].astype(jnp.bfloat16), fk,
                    preferred_element_type=jnp.float32)
        f = fk_ref[pl.ds(m * 2 * hm + i * hm, hm), :]
        out_ref[pl.ds(i * hm, hm), :] = (
            jnp.dot((f + x).astype(jnp.bfloat16), w1t,
                    preferred_element_type=jnp.float32)
            + jnp.dot((x * f).astype(jnp.bfloat16), w2t,
                      preferred_element_type=jnp.float32)
            + b
        )


def kernel(lap_matrix, eye_matrix, features, W1, b1, W2, b2):
    del eye_matrix
    n, d = features.shape
    bm = 400
    hm = bm // 2

    feat_bf = features.astype(jnp.bfloat16)
    w1t = W1.T.astype(jnp.bfloat16)
    w2t = W2.T.astype(jnp.bfloat16)
    bias = (b1 + b2).reshape(1, d)

    grid = (n // bm,)
    return pl.pallas_call(
        _body,
        grid=grid,
        in_specs=[
            pl.BlockSpec((hm, n), lambda m: (2 * m, 0)),
            pl.BlockSpec((hm, n), lambda m: (2 * m + 1, 0)),
            pl.BlockSpec((n, d), lambda m: (0, 0)),
            pl.BlockSpec((d, d), lambda m: (0, 0)),
            pl.BlockSpec((d, d), lambda m: (0, 0)),
            pl.BlockSpec((1, d), lambda m: (0, 0)),
        ],
        out_specs=pl.BlockSpec((bm, d), lambda m: (m, 0)),
        out_shape=jax.ShapeDtypeStruct((n, d), jnp.float32),
        compiler_params=pltpu.CompilerParams(
            dimension_semantics=("arbitrary",),
        ),
    )(lap_matrix, lap_matrix, feat_bf, w1t, w2t, bias)
